# baseline (speedup 1.0000x reference)
import jax
import jax.numpy as jnp
from jax import lax
from jax.experimental import pallas as pl
from jax.experimental.pallas import tpu as pltpu
from jax.experimental.pallas import tpu_sc as plsc

_M = 1_000_000
_D = 15
_B = 16_384
_NB = 2 * _B
_NC = 2
_NS = 16
_NW = _NC * _NS
_NG = _M // 8
_CR = 1024
_NFULL = 30
_LK = 128
_GROWS = _NB // _LK + 2
_PCAP = 2048
_PROWS = _PCAP // _LK
_UPD_W = 16
_SCAN_CHUNK = 4096
_NSCAN = _NB // _SCAN_CHUNK


def _body(mem, idx, upd, out, idxb, gpk, cpk, cpos, pay, buf0, buf1,
          s_i0, s_i1, s_o0, s_o1, s_g):
    wid = lax.axis_index("c") * _NS + lax.axis_index("s")
    lo = ((wid * _NG) // _NW) * 8
    hi = (((wid + 1) * _NG) // _NW) * 8
    rpw = hi - lo
    iot = lax.iota(jnp.int32, 16)

    bufs = (buf0, buf1)
    sin = (s_i0, s_i1)
    sout = (s_o0, s_o1)

    def cbase(ci):
        return jnp.where(ci < _NFULL, ci * _CR, rpw - _CR)

    def cp_in(ci, b):
        return pltpu.make_async_copy(
            mem.at[pl.ds(lo + cbase(ci), _CR)], bufs[b], sin[b])

    def cp_out(ci, b):
        return pltpu.make_async_copy(
            bufs[b], out.at[pl.ds(lo + cbase(ci), _CR)], sout[b])

    cp_in(0, 0).start()

    def memset_body(i, z):
        cpos[i >> 3, pl.ds((i & 7) * 16, 16)] = jnp.zeros((16,), jnp.int32)
        return z
    lax.fori_loop(0, _PROWS * 8, memset_body, 0)

    def scan_chunk(c, cnt_v):
        pltpu.sync_copy(idx.at[pl.ds(c * _SCAN_CHUNK, _SCAN_CHUNK)], idxb)

        def it(i, cnt_v):
            v = idxb[pl.ds(i * 16, 16)]
            m = (v >= lo) & (v < hi)
            pc_v = plsc.all_reduce_population_count(m)
            inc = plsc.cumsum(m.astype(jnp.int32))
            p = cnt_v + inc - 1
            posv = (c * _SCAN_CHUNK) + i * 16 + iot
            e = ((v - lo) << 16) | posv
            plsc.store_scatter(gpk, [p >> 7, p & (_LK - 1)], e, mask=m)
            return cnt_v + pc_v

        return lax.fori_loop(0, _SCAN_CHUNK // 16, it, cnt_v)

    cnt_v = jnp.zeros((16,), jnp.int32)
    for c in range(_NSCAN):
        cnt_v = scan_chunk(c, cnt_v)
    cnt = cnt_v[0]
    nwv = (cnt + 15) >> 4

    def walk(cb, ce, p):
        cbp = cb << 16
        cep = ce << 16
        pbase = p * _PCAP

        def it(w, kv):
            gv = gpk[w >> 3, pl.ds((w & 7) * 16, 16)]
            mw = (w * 16 + iot) < cnt_v
            m2 = mw & (gv >= cbp) & (gv < cep)
            inc = plsc.cumsum(m2.astype(jnp.int32))
            ordv = kv + inc - 1
            mp = m2 & (ordv >= pbase) & (ordv < pbase + _PCAP)
            q = ordv - pbase
            plsc.store_scatter(cpk, [q >> 7, q & (_LK - 1)], gv, mask=mp)
            plsc.store_scatter(cpos, [q >> 7, q & (_LK - 1)], gv & 0xFFFF,
                               mask=mp)
            return kv + plsc.all_reduce_population_count(m2)

        return lax.fori_loop(0, nwv, it, jnp.zeros((16,), jnp.int32))[0]

    def process(ci, buf):
        cb = cbase(ci)
        ce = cb + _CR

        def pass_body(carry):
            p, _ = carry
            kc = walk(cb, ce, p)
            kp = jnp.clip(kc - p * _PCAP, 0, _PCAP)
            ngr = (kp + (_LK - 1)) >> 7

            def group(g, z2):
                gather = pltpu.make_async_copy(upd.at[cpos.at[g]], pay, s_g)
                gather.start()
                gather.wait()
                kg = jnp.clip(kp - g * _LK, 0, _LK)
                for vi in range(_LK // 16):
                    ev = cpk[g, pl.ds(vi * 16, 16)]
                    for l in range(16):
                        @pl.when(vi * 16 + l < kg)
                        def _():
                            brow = (ev[l] >> 16) - cb
                            pvec = pay[vi * 16 + l, pl.ds(0, 16)]
                            plsc.store_scatter(
                                buf, [jnp.broadcast_to(brow, (16,)), iot],
                                pvec, mask=iot < _D)
                return z2

            lax.fori_loop(0, ngr, group, 0)
            return (p + 1, kc)

        lax.while_loop(lambda c: c[0] * _PCAP < c[1], pass_body,
                       (jnp.int32(0), jnp.int32(1)))

    def half(ci, bx, guard_prev):
        cp_in(ci, bx).wait()
        if guard_prev is None:
            cp_out(ci - 1, 1 - bx).wait()
            cp_in(ci + 1, 1 - bx).start()
        else:
            @pl.when(guard_prev)
            def _():
                cp_out(ci - 1, 1 - bx).wait()
            cp_in(ci + 1, 1 - bx).start()
        process(ci, bufs[bx])
        cp_out(ci, bx).start()

    def pair(j, z):
        half(2 * j, 0, j > 0)
        half(2 * j + 1, 1, None)
        return z

    lax.fori_loop(0, _NFULL // 2, pair, 0)
    cp_in(_NFULL, 0).wait()
    process(jnp.int32(_NFULL), bufs[0])
    cp_out(_NFULL, 0).start()
    cp_out(_NFULL, 0).wait()
    cp_out(_NFULL - 1, 1).wait()


@jax.jit
def _run(mem, idx, upd):
    f = pl.kernel(
        _body,
        out_type=jax.ShapeDtypeStruct((_M, _D), jnp.float32),
        mesh=plsc.VectorSubcoreMesh(
            core_axis_name="c", subcore_axis_name="s",
            num_cores=_NC, num_subcores=_NS),
        compiler_params=pltpu.CompilerParams(
            needs_layout_passes=False, use_tc_tiling_on_sc=False),
        scratch_types=[
            pltpu.VMEM((_SCAN_CHUNK,), jnp.int32),
            pltpu.VMEM((_GROWS, _LK), jnp.int32),
            pltpu.VMEM((_PROWS, _LK), jnp.int32),
            pltpu.VMEM((_PROWS, _LK), jnp.int32),
            pltpu.VMEM((_LK, _UPD_W), jnp.float32),
            pltpu.VMEM((_CR, _D), jnp.float32),
            pltpu.VMEM((_CR, _D), jnp.float32),
            pltpu.SemaphoreType.DMA,
            pltpu.SemaphoreType.DMA,
            pltpu.SemaphoreType.DMA,
            pltpu.SemaphoreType.DMA,
            pltpu.SemaphoreType.DMA,
        ],
    )
    return f(mem, idx, upd)


def kernel(mem, srcID, src_feature, dstID, dst_feature):
    idx = jnp.concatenate([srcID, dstID], axis=0)
    upd = jnp.pad(jnp.concatenate([src_feature, dst_feature], axis=0),
                  ((0, 0), (0, _UPD_W - _D)))
    return _run(mem, idx, upd)


# speedup vs baseline: 1.0046x; 1.0046x over previous
import jax
import jax.numpy as jnp
from jax import lax
from jax.experimental import pallas as pl
from jax.experimental.pallas import tpu as pltpu
from jax.experimental.pallas import tpu_sc as plsc

_M = 1_000_000
_D = 15
_B = 16_384
_NB = 2 * _B
_NC = 2
_NS = 16
_NW = _NC * _NS
_NG = _M // 8
_CR = 1024
_NFULL = 30
_LK = 128
_GROWS = _NB // _LK + 2
_PCAP = 2048
_PROWS = _PCAP // _LK
_UPD_W = 16
_SCAN_CHUNK = 4096
_NSCAN = _NB // _SCAN_CHUNK


def _body(mem, idx, upd, out, idxb, gpk, cpk, cpos, pay, buf0, buf1,
          s_i0, s_i1, s_o0, s_o1, s_g):
    wid = lax.axis_index("c") * _NS + lax.axis_index("s")
    lo = ((wid * _NG) // _NW) * 8
    hi = (((wid + 1) * _NG) // _NW) * 8
    rpw = hi - lo
    iot = lax.iota(jnp.int32, 16)

    bufs = (buf0, buf1)
    sin = (s_i0, s_i1)
    sout = (s_o0, s_o1)

    def cbase(ci):
        return jnp.where(ci < _NFULL, ci * _CR, rpw - _CR)

    def cp_in(ci, b):
        return pltpu.make_async_copy(
            mem.at[pl.ds(lo + cbase(ci), _CR)], bufs[b], sin[b])

    def cp_out(ci, b):
        return pltpu.make_async_copy(
            bufs[b], out.at[pl.ds(lo + cbase(ci), _CR)], sout[b])

    cp_in(0, 0).start()

    def memset_body(i, z):
        cpos[i >> 3, pl.ds((i & 7) * 16, 16)] = jnp.zeros((16,), jnp.int32)
        return z
    lax.fori_loop(0, _PROWS * 8, memset_body, 0)

    def scan_chunk(c, cnt_v):
        pltpu.sync_copy(idx.at[pl.ds(c * _SCAN_CHUNK, _SCAN_CHUNK)], idxb)

        def it(i, cnt_v):
            v = idxb[pl.ds(i * 16, 16)]
            m = (v >= lo) & (v < hi)
            pc_v = plsc.all_reduce_population_count(m)
            inc = plsc.cumsum(m.astype(jnp.int32))
            p = cnt_v + inc - 1
            posv = (c * _SCAN_CHUNK) + i * 16 + iot
            e = ((v - lo) << 16) | posv
            plsc.store_scatter(gpk, [p >> 7, p & (_LK - 1)], e, mask=m)
            return cnt_v + pc_v

        return lax.fori_loop(0, _SCAN_CHUNK // 16, it, cnt_v)

    cnt_v = jnp.zeros((16,), jnp.int32)
    for c in range(_NSCAN):
        cnt_v = scan_chunk(c, cnt_v)
    cnt = cnt_v[0]
    nwv = (cnt + 15) >> 4

    def walk(cb, ce, p):
        cbp = cb << 16
        cep = ce << 16
        pbase = p * _PCAP

        def it(w, kv):
            gv = gpk[w >> 3, pl.ds((w & 7) * 16, 16)]
            mw = (w * 16 + iot) < cnt_v
            m2 = mw & (gv >= cbp) & (gv < cep)
            inc = plsc.cumsum(m2.astype(jnp.int32))
            ordv = kv + inc - 1
            mp = m2 & (ordv >= pbase) & (ordv < pbase + _PCAP)
            q = ordv - pbase
            plsc.store_scatter(cpk, [q >> 7, q & (_LK - 1)], gv, mask=mp)
            plsc.store_scatter(cpos, [q >> 7, q & (_LK - 1)], gv & 0xFFFF,
                               mask=mp)
            return kv + plsc.all_reduce_population_count(m2)

        return lax.fori_loop(0, nwv, it, jnp.zeros((16,), jnp.int32))[0]

    def process(ci, buf):
        cb = cbase(ci)
        ce = cb + _CR

        def pass_body(carry):
            p, _ = carry
            kc = walk(cb, ce, p)
            kp = jnp.clip(kc - p * _PCAP, 0, _PCAP)
            ngr = (kp + (_LK - 1)) >> 7

            def group(g, z2):
                gather = pltpu.make_async_copy(upd.at[cpos.at[g]], pay, s_g)
                gather.start()
                gather.wait()
                kg = jnp.clip(kp - g * _LK, 0, _LK)

                def vloop(vi, z3):
                    ev = cpk[g, pl.ds(vi * 16, 16)]
                    for l in range(16):
                        @pl.when(vi * 16 + l < kg)
                        def _():
                            brow = (ev[l] >> 16) - cb
                            pvec = pay[vi * 16 + l, pl.ds(0, 16)]
                            plsc.store_scatter(
                                buf, [jnp.broadcast_to(brow, (16,)), iot],
                                pvec, mask=iot < _D)
                    return z3

                lax.fori_loop(0, (kg + 15) >> 4, vloop, 0)
                return z2

            lax.fori_loop(0, ngr, group, 0)
            return (p + 1, kc)

        lax.while_loop(lambda c: c[0] * _PCAP < c[1], pass_body,
                       (jnp.int32(0), jnp.int32(1)))

    def do_chunk(ci, z):
        cp_in(ci, 0).wait()
        process(ci, buf0)
        cp_out(ci, 0).start()

        @pl.when(ci < _NFULL)
        def _():
            cp_out(ci, 0).wait()
            cp_in(ci + 1, 0).start()
        return z

    lax.fori_loop(0, _NFULL + 1, do_chunk, 0)
    cp_out(_NFULL, 0).wait()


@jax.jit
def _run(mem, idx, upd):
    f = pl.kernel(
        _body,
        out_type=jax.ShapeDtypeStruct((_M, _D), jnp.float32),
        mesh=plsc.VectorSubcoreMesh(
            core_axis_name="c", subcore_axis_name="s",
            num_cores=_NC, num_subcores=_NS),
        compiler_params=pltpu.CompilerParams(
            needs_layout_passes=False, use_tc_tiling_on_sc=False),
        scratch_types=[
            pltpu.VMEM((_SCAN_CHUNK,), jnp.int32),
            pltpu.VMEM((_GROWS, _LK), jnp.int32),
            pltpu.VMEM((_PROWS, _LK), jnp.int32),
            pltpu.VMEM((_PROWS, _LK), jnp.int32),
            pltpu.VMEM((_LK, _UPD_W), jnp.float32),
            pltpu.VMEM((_CR, _D), jnp.float32),
            pltpu.VMEM((_CR, _D), jnp.float32),
            pltpu.SemaphoreType.DMA,
            pltpu.SemaphoreType.DMA,
            pltpu.SemaphoreType.DMA,
            pltpu.SemaphoreType.DMA,
            pltpu.SemaphoreType.DMA,
        ],
    )
    return f(mem, idx, upd)


def kernel(mem, srcID, src_feature, dstID, dst_feature):
    idx = jnp.concatenate([srcID, dstID], axis=0)
    upd = jnp.pad(jnp.concatenate([src_feature, dst_feature], axis=0),
                  ((0, 0), (0, _UPD_W - _D)))
    return _run(mem, idx, upd)
